# Initial kernel scaffold; baseline (speedup 1.0000x reference)
import functools

import jax
import jax.numpy as jnp
from jax import lax
from jax.experimental import pallas as pl
from jax.experimental.pallas import tpu as pltpu
from jax.experimental.pallas import tpu_sc as plsc

D = 32
NUM_CORES = 2
NUM_SUBCORES = 16
NW = NUM_CORES * NUM_SUBCORES
CHUNK = 1024


def _make_embed(b_total: int):
    b_per_w = b_total // NW
    n_chunks = b_per_w // CHUNK
    mesh = plsc.VectorSubcoreMesh(core_axis_name="c", subcore_axis_name="s")

    @functools.partial(
        pl.kernel,
        mesh=mesh,
        out_type=jax.ShapeDtypeStruct((b_total, D), jnp.float32),
        scratch_types=[
            pltpu.VMEM((CHUNK,), jnp.int32),
            pltpu.VMEM((CHUNK, D), jnp.float32),
            pltpu.SemaphoreType.DMA,
        ],
    )
    def embed(table_hbm, idx_hbm, out_hbm, idx_v, rows_v, sem):
        wid = lax.axis_index("s") * NUM_CORES + lax.axis_index("c")
        base = wid * b_per_w

        def body(i, carry):
            off = base + i * CHUNK
            pltpu.sync_copy(idx_hbm.at[pl.ds(off, CHUNK)], idx_v)
            pltpu.async_copy(table_hbm.at[idx_v], rows_v, sem).wait()
            pltpu.sync_copy(rows_v, out_hbm.at[pl.ds(off, CHUNK)])
            return carry

        lax.fori_loop(0, n_chunks, body, 0)

    return embed


def kernel(n_flat, embedding):
    batch, fields = n_flat.shape
    idx = n_flat.reshape(-1).astype(jnp.int32)
    out = _make_embed(batch * fields)(embedding, idx)
    return out.reshape(batch, fields, D)


# speedup vs baseline: 1.6284x; 1.6284x over previous
import functools

import jax
import jax.numpy as jnp
from jax import lax
from jax.experimental import pallas as pl
from jax.experimental.pallas import tpu as pltpu
from jax.experimental.pallas import tpu_sc as plsc

D = 32
NUM_CORES = 2
NUM_SUBCORES = 16
NW = NUM_CORES * NUM_SUBCORES
CHUNK = 1024


def _make_embed(b_total: int):
    b_per_w = b_total // NW
    n_chunks = b_per_w // CHUNK
    mesh = plsc.VectorSubcoreMesh(core_axis_name="c", subcore_axis_name="s")

    @functools.partial(
        pl.kernel,
        mesh=mesh,
        out_type=jax.ShapeDtypeStruct((b_total, D), jnp.float32),
        scratch_types=[
            pltpu.VMEM((CHUNK,), jnp.int32),
            pltpu.VMEM((CHUNK, D), jnp.float32),
            pltpu.SemaphoreType.DMA,
        ],
        compiler_params=pltpu.CompilerParams(use_tc_tiling_on_sc=False),
    )
    def embed(table_hbm, idx_hbm, out_hbm, idx_v, rows_v, sem):
        wid = lax.axis_index("s") * NUM_CORES + lax.axis_index("c")
        base = wid * b_per_w

        def body(i, carry):
            off = base + i * CHUNK
            pltpu.sync_copy(idx_hbm.at[pl.ds(off, CHUNK)], idx_v)
            pltpu.async_copy(table_hbm.at[idx_v], rows_v, sem).wait()
            pltpu.sync_copy(rows_v, out_hbm.at[pl.ds(off, CHUNK)])
            return carry

        lax.fori_loop(0, n_chunks, body, 0)

    return embed


def kernel(n_flat, embedding):
    batch, fields = n_flat.shape
    idx = n_flat.reshape(-1).astype(jnp.int32)
    out = _make_embed(batch * fields)(embedding, idx)
    return out.reshape(batch, fields, D)


# speedup vs baseline: 1.6504x; 1.0135x over previous
import functools

import jax
import jax.numpy as jnp
from jax import lax
from jax.experimental import pallas as pl
from jax.experimental.pallas import tpu as pltpu
from jax.experimental.pallas import tpu_sc as plsc

D = 32
NUM_CORES = 2
NUM_SUBCORES = 16
NW = NUM_CORES * NUM_SUBCORES
CHUNK = 1664
NBUF = 2


def _make_embed(b_total: int):
    b_per_w = b_total // NW
    n_chunks = b_per_w // CHUNK
    mesh = plsc.VectorSubcoreMesh(core_axis_name="c", subcore_axis_name="s")

    @functools.partial(
        pl.kernel,
        mesh=mesh,
        out_type=jax.ShapeDtypeStruct((b_total, D), jnp.float32),
        scratch_types=[
            pltpu.VMEM((CHUNK,), jnp.int32),
            pltpu.VMEM((CHUNK,), jnp.int32),
            pltpu.VMEM((CHUNK, D), jnp.float32),
            pltpu.VMEM((CHUNK, D), jnp.float32),
            pltpu.SemaphoreType.DMA,
            pltpu.SemaphoreType.DMA,
            pltpu.SemaphoreType.DMA,
            pltpu.SemaphoreType.DMA,
            pltpu.SemaphoreType.DMA,
        ],
        compiler_params=pltpu.CompilerParams(use_tc_tiling_on_sc=False),
    )
    def embed(
        table_hbm, idx_hbm, out_hbm, ix0, ix1, rw0, rw1, is0, is1, gs, os0, os1
    ):
        wid = lax.axis_index("s") * NUM_CORES + lax.axis_index("c")
        base = wid * b_per_w
        idx_v = (ix0, ix1)
        rows_v = (rw0, rw1)
        isem = (is0, is1)
        osem = (os0, os1)

        icopy = [
            pltpu.async_copy(
                idx_hbm.at[pl.ds(base + b * CHUNK, CHUNK)], idx_v[b], isem[b]
            )
            for b in range(NBUF)
        ]
        ocopy = [None] * NBUF
        for i in range(n_chunks):
            b = i % NBUF
            icopy[b].wait()
            if ocopy[b] is not None:
                ocopy[b].wait()
            pltpu.async_copy(table_hbm.at[idx_v[b]], rows_v[b], gs).wait()
            if i + NBUF < n_chunks:
                icopy[b] = pltpu.async_copy(
                    idx_hbm.at[pl.ds(base + (i + NBUF) * CHUNK, CHUNK)],
                    idx_v[b],
                    isem[b],
                )
            ocopy[b] = pltpu.async_copy(
                rows_v[b], out_hbm.at[pl.ds(base + i * CHUNK, CHUNK)], osem[b]
            )
        for b in range(NBUF):
            if ocopy[b] is not None:
                ocopy[b].wait()

    return embed


def kernel(n_flat, embedding):
    batch, fields = n_flat.shape
    idx = n_flat.reshape(-1).astype(jnp.int32)
    out = _make_embed(batch * fields)(embedding, idx)
    return out.reshape(batch, fields, D)


# speedup vs baseline: 1.7630x; 1.0682x over previous
import functools

import jax
import jax.numpy as jnp
from jax import lax
from jax.experimental import pallas as pl
from jax.experimental.pallas import tpu as pltpu
from jax.experimental.pallas import tpu_sc as plsc

D = 32
NUM_CORES = 2
NUM_SUBCORES = 16
NW = NUM_CORES * NUM_SUBCORES
CHUNK = 1664
NBUF = 2


def _make_embed(b_total: int):
    b_per_w = b_total // NW
    n_chunks = b_per_w // CHUNK
    mesh = plsc.VectorSubcoreMesh(core_axis_name="c", subcore_axis_name="s")

    @functools.partial(
        pl.kernel,
        mesh=mesh,
        out_type=jax.ShapeDtypeStruct((b_total, D), jnp.float32),
        scratch_types=[
            pltpu.VMEM((CHUNK,), jnp.int32),
            pltpu.VMEM((CHUNK,), jnp.int32),
            pltpu.VMEM((CHUNK, D), jnp.float32),
            pltpu.VMEM((CHUNK, D), jnp.float32),
            pltpu.SemaphoreType.DMA,
            pltpu.SemaphoreType.DMA,
            pltpu.SemaphoreType.DMA,
            pltpu.SemaphoreType.DMA,
            pltpu.SemaphoreType.DMA,
        ],
        compiler_params=pltpu.CompilerParams(use_tc_tiling_on_sc=False),
    )
    def embed(
        table_hbm, idx_hbm, out_hbm, ix0, ix1, rw0, rw1, is0, is1, gs, os0, os1
    ):
        wid = lax.axis_index("s") * NUM_CORES + lax.axis_index("c")
        base = wid * b_per_w
        idx_v = (ix0, ix1)
        rows_v = (rw0, rw1)
        isem = (is0, is1)
        osem = (os0, os1)

        icopy = [
            pltpu.async_copy(
                idx_hbm.at[pl.ds(base + b * CHUNK, CHUNK)], idx_v[b], isem[b]
            )
            for b in range(NBUF)
        ]
        ocopy = [None] * NBUF
        for i in range(n_chunks):
            b = i % NBUF
            icopy[b].wait()
            if ocopy[b] is not None:
                ocopy[b].wait()
            pltpu.async_copy(table_hbm.at[idx_v[b]], rows_v[b], gs).wait()
            if i + NBUF < n_chunks:
                icopy[b] = pltpu.async_copy(
                    idx_hbm.at[pl.ds(base + (i + NBUF) * CHUNK, CHUNK)],
                    idx_v[b],
                    isem[b],
                )
            ocopy[b] = pltpu.async_copy(
                rows_v[b], out_hbm.at[pl.ds(base + i * CHUNK, CHUNK)], osem[b]
            )
        for b in range(NBUF):
            if ocopy[b] is not None:
                ocopy[b].wait()

    return embed


def kernel(n_flat, embedding):
    batch, fields = n_flat.shape
    idx_fm = n_flat.T.reshape(-1).astype(jnp.int32)
    out2 = _make_embed(batch * fields)(embedding, idx_fm)
    return out2.reshape(fields, batch, D).transpose(1, 0, 2)
